# speedup vs baseline: 1.2994x
import functools

import jax
import jax.numpy as jnp
from jax import lax
from jax.experimental import pallas as pl
from jax.experimental.pallas import tpu as pltpu
from jax.experimental.pallas import tpu_sc as plsc

N = 10000
M = 128
H = 1280
E = 160000
CH = H // M

NC = 2
NS = 16
NW = NC * NS

PN = 10240
B = 128
EPW = 5120
NB = EPW // B
EP = NW * EPW


def _make_seg_sum(nch, tbl_rows):
    mesh = plsc.VectorSubcoreMesh(
        core_axis_name="c", subcore_axis_name="s",
        num_cores=NC, num_subcores=NS)

    @functools.partial(
        pl.kernel,
        out_type=jax.ShapeDtypeStruct((NC, nch, PN, M), jnp.float32),
        mesh=mesh,
        scratch_types=[
            pltpu.VMEM_SHARED((PN, M), jnp.float32),
            pltpu.VMEM((NB, B), jnp.int32),
            pltpu.VMEM((NB, B), jnp.int32),
            pltpu.VMEM((B, M), jnp.float32),
            pltpu.SemaphoreType.DMA,
        ],
    )
    def k(tbl, srcp, dstp, zeros, out, acc, isrc, idst, rows, sem):
        cid = lax.axis_index("c")
        sid = lax.axis_index("s")
        wid = sid * NC + cid
        pltpu.sync_copy(srcp.at[pl.ds(wid * NB, NB)], isrc)
        pltpu.sync_copy(dstp.at[pl.ds(wid * NB, NB)], idst)
        rps = PN // NS
        zbase = sid * rps

        def chunk_body(c, carry):
            pltpu.sync_copy(zeros.at[pl.ds(zbase, rps)],
                            acc.at[pl.ds(zbase, rps)])
            plsc.subcore_barrier()

            def step(b, carry2):
                pltpu.async_copy(tbl.at[c].at[isrc.at[b]], rows, sem).wait()
                pltpu.sync_copy(rows, acc.at[idst.at[b]], add=True)
                return carry2

            lax.fori_loop(0, NB, step, 0)
            plsc.subcore_barrier()
            pltpu.sync_copy(acc.at[pl.ds(zbase, rps)],
                            out.at[cid, c, pl.ds(zbase, rps)])
            plsc.subcore_barrier()
            return carry

        lax.fori_loop(0, nch, chunk_body, 0)

    return k


TR0 = 256
TR1 = 400


def _mlp0_body(p_ref, w0_ref, b0_ref, w1_ref, b1_ref, w2_ref, b2_ref, out_ref):
    agg = p_ref[0, 0] + p_ref[1, 0]
    t = jnp.dot(agg, w0_ref[...], preferred_element_type=jnp.float32)
    t = jnp.maximum(t + b0_ref[...], 0.0)
    t = jnp.dot(t, w1_ref[...], preferred_element_type=jnp.float32)
    t = jnp.maximum(t + b1_ref[...], 0.0)
    t = jnp.dot(t, w2_ref[...], preferred_element_type=jnp.float32)
    t = jnp.maximum(t + b2_ref[...], 0.0)
    for c in range(CH):
        out_ref[c] = t[:, c * M:(c + 1) * M]


def _mlp0(p0, W0, b0, W1, b1, W2, b2):
    return pl.pallas_call(
        _mlp0_body,
        grid=(PN // TR0,),
        in_specs=[
            pl.BlockSpec((NC, 1, TR0, M), lambda r: (0, 0, r, 0)),
            pl.BlockSpec((M, H), lambda r: (0, 0)),
            pl.BlockSpec((1, H), lambda r: (0, 0)),
            pl.BlockSpec((H, H), lambda r: (0, 0)),
            pl.BlockSpec((1, H), lambda r: (0, 0)),
            pl.BlockSpec((H, H), lambda r: (0, 0)),
            pl.BlockSpec((1, H), lambda r: (0, 0)),
        ],
        out_specs=pl.BlockSpec((CH, TR0, M), lambda r: (0, r, 0)),
        out_shape=jax.ShapeDtypeStruct((CH, PN, M), jnp.float32),
    )(p0, W0, b0, W1, b1, W2, b2)


def _mlp1_body(p_ref, w0_ref, b0_ref, w1_ref, b1_ref, w2_ref, b2_ref, out_ref):
    agg = jnp.concatenate(
        [p_ref[0, c] + p_ref[1, c] for c in range(CH)], axis=1)
    t = jnp.dot(agg, w0_ref[...], preferred_element_type=jnp.float32)
    t = jnp.maximum(t + b0_ref[...], 0.0)
    t = jnp.dot(t, w1_ref[...], preferred_element_type=jnp.float32)
    t = jnp.maximum(t + b1_ref[...], 0.0)
    t = jnp.dot(t, w2_ref[...], preferred_element_type=jnp.float32)
    t = t + b2_ref[...]
    m = jnp.max(t, axis=1, keepdims=True)
    ex = jnp.exp(t - m)
    s = jnp.sum(ex, axis=1, keepdims=True)
    out_ref[...] = t - m - jnp.log(s)


def _mlp1(p1, W0, b0, W1, b1, W2, b2):
    return pl.pallas_call(
        _mlp1_body,
        grid=(N // TR1,),
        in_specs=[
            pl.BlockSpec((NC, CH, TR1, M), lambda r: (0, 0, r, 0)),
            pl.BlockSpec((H, H), lambda r: (0, 0)),
            pl.BlockSpec((1, H), lambda r: (0, 0)),
            pl.BlockSpec((H, H), lambda r: (0, 0)),
            pl.BlockSpec((1, H), lambda r: (0, 0)),
            pl.BlockSpec((H, H), lambda r: (0, 0)),
            pl.BlockSpec((1, H), lambda r: (0, 0)),
        ],
        out_specs=pl.BlockSpec((TR1, H), lambda r: (r, 0)),
        out_shape=jax.ShapeDtypeStruct((N, H), jnp.float32),
    )(p1, W0, b0, W1, b1, W2, b2)


def kernel(x, edge_index, W0_0, b0_0, W0_1, b0_1, W0_2, b0_2,
           W1_0, b1_0, W1_1, b1_1, W1_2, b1_2):
    src = edge_index[0]
    dst = edge_index[1]
    pad = EP - E
    srcp = jnp.concatenate(
        [src, jnp.zeros((pad,), jnp.int32)]).reshape(EP // B, B)
    dstp = jnp.concatenate(
        [dst, jnp.full((pad,), N, jnp.int32)]).reshape(EP // B, B)
    zeros = jnp.zeros((PN, M), jnp.float32)

    p0 = _make_seg_sum(1, N)(x.reshape(1, N, M), srcp, dstp, zeros)
    h1 = _mlp0(p0, W0_0, b0_0.reshape(1, H), W0_1, b0_1.reshape(1, H),
               W0_2, b0_2.reshape(1, H))
    p1 = _make_seg_sum(CH, PN)(h1, srcp, dstp, zeros)
    return _mlp1(p1, W1_0, b1_0.reshape(1, H), W1_1, b1_1.reshape(1, H),
                 W1_2, b1_2.reshape(1, H))


# speedup vs baseline: 1.9240x; 1.9240x over previous
import functools

import jax
import jax.numpy as jnp
from jax import lax
from jax.experimental import pallas as pl
from jax.experimental.pallas import tpu as pltpu
from jax.experimental.pallas import tpu_sc as plsc

N = 10000
M = 128
H = 1280
E = 160000
CH = H // M

NC = 2
NS = 16
NW = NC * NS

PN = 10112
B = 96
NB0 = 88
NB1 = 18
E0 = NS * NB0 * B
E1 = NS * NB1 * B
EP = E0 + E1


def _make_seg_sum(nch, tbl_rows):
    mesh = plsc.VectorSubcoreMesh(
        core_axis_name="c", subcore_axis_name="s",
        num_cores=NC, num_subcores=NS)

    @functools.partial(
        pl.kernel,
        out_type=jax.ShapeDtypeStruct((NC, nch, PN, M), jnp.float32),
        mesh=mesh,
        scratch_types=(
            [pltpu.VMEM_SHARED((PN, M), jnp.float32)]
            + [pltpu.VMEM((NB0, B), jnp.int32)] * 2
            + [pltpu.VMEM((B, M), jnp.float32)] * 2
            + [pltpu.SemaphoreType.DMA] * 2
        ),
    )
    def k(tbl, s0, d0, s1, d1, out, acc, isrc, idst, *rest):
        rows = rest[:2]
        gs = rest[2:4]
        cid = lax.axis_index("c")
        sid = lax.axis_index("s")

        @pl.when(cid == 0)
        def _():
            pltpu.sync_copy(s0.at[sid], isrc)
            pltpu.sync_copy(d0.at[sid], idst)

        @pl.when(cid == 1)
        def _():
            pltpu.sync_copy(s1.at[sid], isrc.at[pl.ds(0, NB1)])
            pltpu.sync_copy(d1.at[sid], idst.at[pl.ds(0, NB1)])

        nbw = lax.select(cid == 0, NB0, NB1)
        rps = PN // NS
        zbase = sid * rps

        zer = jnp.zeros((16,), jnp.float32)

        def chunk_body(c, carry):
            def zrow(r, carry2):
                for j in range(M // 16):
                    rows[0][r, pl.ds(j * 16, 16)] = zer
                return carry2

            lax.fori_loop(0, B, zrow, 0)
            for i in range(rps // B):
                pltpu.sync_copy(rows[0],
                                acc.at[pl.ds(zbase + i * B, B)])
            rem = rps % B
            if rem:
                pltpu.sync_copy(rows[0].at[pl.ds(0, rem)],
                                acc.at[pl.ds(zbase + (rps // B) * B, rem)])
            plsc.subcore_barrier()
            for kk in range(2):
                pltpu.async_copy(tbl.at[c].at[isrc.at[kk]], rows[kk], gs[kk])

            def group(g, carry2):
                for kk in range(2):
                    b = g * 2 + kk
                    pltpu.make_async_copy(
                        tbl.at[c].at[isrc.at[b]], rows[kk], gs[kk]).wait()
                    pltpu.sync_copy(rows[kk], acc.at[idst.at[b]], add=True)
                    nb = b + 2

                    @pl.when(nb < nbw)
                    def _():
                        pltpu.async_copy(
                            tbl.at[c].at[isrc.at[nb]], rows[kk], gs[kk])
                return carry2

            lax.fori_loop(0, nbw // 2, group, 0)
            plsc.subcore_barrier()
            pltpu.sync_copy(acc.at[pl.ds(zbase, rps)],
                            out.at[cid, c, pl.ds(zbase, rps)])
            plsc.subcore_barrier()
            return carry

        lax.fori_loop(0, nch, chunk_body, 0)

    return k


TR0 = 632
TR1 = 400


def _mlp0_body(p_ref, w0_ref, b0_ref, w1_ref, b1_ref, w2_ref, b2_ref, out_ref):
    agg = p_ref[0, 0] + p_ref[1, 0]
    t = jnp.dot(agg, w0_ref[...], preferred_element_type=jnp.float32)
    t = jnp.maximum(t + b0_ref[...], 0.0)
    t = jnp.dot(t, w1_ref[...], preferred_element_type=jnp.float32)
    t = jnp.maximum(t + b1_ref[...], 0.0)
    t = jnp.dot(t, w2_ref[...], preferred_element_type=jnp.float32)
    t = jnp.maximum(t + b2_ref[...], 0.0)
    for c in range(CH):
        out_ref[c] = t[:, c * M:(c + 1) * M]


def _mlp0(p0, W0, b0, W1, b1, W2, b2):
    return pl.pallas_call(
        _mlp0_body,
        grid=(PN // TR0,),
        in_specs=[
            pl.BlockSpec((NC, 1, TR0, M), lambda r: (0, 0, r, 0)),
            pl.BlockSpec((M, H), lambda r: (0, 0)),
            pl.BlockSpec((1, H), lambda r: (0, 0)),
            pl.BlockSpec((H, H), lambda r: (0, 0)),
            pl.BlockSpec((1, H), lambda r: (0, 0)),
            pl.BlockSpec((H, H), lambda r: (0, 0)),
            pl.BlockSpec((1, H), lambda r: (0, 0)),
        ],
        out_specs=pl.BlockSpec((CH, TR0, M), lambda r: (0, r, 0)),
        out_shape=jax.ShapeDtypeStruct((CH, PN, M), jnp.float32),
    )(p0, W0, b0, W1, b1, W2, b2)


def _mlp1_body(p_ref, w0_ref, b0_ref, w1_ref, b1_ref, w2_ref, b2_ref, out_ref):
    agg = jnp.concatenate(
        [p_ref[0, c] + p_ref[1, c] for c in range(CH)], axis=1)
    t = jnp.dot(agg, w0_ref[...], preferred_element_type=jnp.float32)
    t = jnp.maximum(t + b0_ref[...], 0.0)
    t = jnp.dot(t, w1_ref[...], preferred_element_type=jnp.float32)
    t = jnp.maximum(t + b1_ref[...], 0.0)
    t = jnp.dot(t, w2_ref[...], preferred_element_type=jnp.float32)
    t = t + b2_ref[...]
    m = jnp.max(t, axis=1, keepdims=True)
    ex = jnp.exp(t - m)
    s = jnp.sum(ex, axis=1, keepdims=True)
    out_ref[...] = t - m - jnp.log(s)


def _mlp1(p1, W0, b0, W1, b1, W2, b2):
    return pl.pallas_call(
        _mlp1_body,
        grid=(N // TR1,),
        in_specs=[
            pl.BlockSpec((NC, CH, TR1, M), lambda r: (0, 0, r, 0)),
            pl.BlockSpec((H, H), lambda r: (0, 0)),
            pl.BlockSpec((1, H), lambda r: (0, 0)),
            pl.BlockSpec((H, H), lambda r: (0, 0)),
            pl.BlockSpec((1, H), lambda r: (0, 0)),
            pl.BlockSpec((H, H), lambda r: (0, 0)),
            pl.BlockSpec((1, H), lambda r: (0, 0)),
        ],
        out_specs=pl.BlockSpec((TR1, H), lambda r: (r, 0)),
        out_shape=jax.ShapeDtypeStruct((N, H), jnp.float32),
    )(p1, W0, b0, W1, b1, W2, b2)


def kernel(x, edge_index, W0_0, b0_0, W0_1, b0_1, W0_2, b0_2,
           W1_0, b1_0, W1_1, b1_1, W1_2, b1_2):
    src = edge_index[0]
    dst = edge_index[1]
    pad = EP - E
    trash = N + jnp.arange(pad, dtype=jnp.int32) % (PN - N)
    srcp = jnp.concatenate([src, jnp.zeros((pad,), jnp.int32)])
    dstp = jnp.concatenate([dst, trash])
    s0 = srcp[:E0].reshape(NS, NB0, B)
    d0 = dstp[:E0].reshape(NS, NB0, B)
    s1 = srcp[E0:].reshape(NS, NB1, B)
    d1 = dstp[E0:].reshape(NS, NB1, B)

    p0 = _make_seg_sum(1, N)(x.reshape(1, N, M), s0, d0, s1, d1)
    h1 = _mlp0(p0, W0_0, b0_0.reshape(1, H), W0_1, b0_1.reshape(1, H),
               W0_2, b0_2.reshape(1, H))
    p1 = _make_seg_sum(CH, PN)(h1, s0, d0, s1, d1)
    return _mlp1(p1, W1_0, b1_0.reshape(1, H), W1_1, b1_1.reshape(1, H),
                 W1_2, b1_2.reshape(1, H))
